# baseline (speedup 1.0000x reference)
import jax
import jax.numpy as jnp
from jax import lax
from jax.experimental import pallas as pl
from jax.experimental.pallas import tpu as pltpu

B, I, F = 4096, 16, 512
CB = 64
NCHUNK = B // CB
NBUF = 8
NOUTER = NCHUNK // NBUF
assert NCHUNK % NBUF == 0


def _ring_kernel(o_hbm, l_hbm, imp_ref, o_ref, *scr):
    obufs = scr[0:NBUF]
    lbufs = scr[NBUF:2 * NBUF]
    osems = scr[2 * NBUF:3 * NBUF]
    lsems = scr[3 * NBUF:4 * NBUF]

    imp = imp_ref[...]

    for s in range(NBUF):
        boff = s * CB
        pltpu.async_copy(o_hbm.at[pl.ds(boff, CB)], obufs[s], osems[s])
        pltpu.async_copy(l_hbm.at[pl.ds(boff, CB)], lbufs[s], lsems[s])

    def outer_body(c0, acc):
        for s in range(NBUF):
            c = c0 * NBUF + s
            boff = c * CB
            ob, lb = obufs[s], lbufs[s]
            pltpu.make_async_copy(o_hbm.at[pl.ds(boff, CB)], ob, osems[s]).wait()
            pltpu.make_async_copy(l_hbm.at[pl.ds(boff, CB)], lb, lsems[s]).wait()

            def b_body(b, a, ob=ob, lb=lb):
                d = imp * (jnp.abs(lb[b]) - ob[b])
                return a + d * d

            acc = lax.fori_loop(0, CB, b_body, acc, unroll=2)

            @pl.when(c0 < NOUTER - 1)
            def _():
                boff2 = boff + NBUF * CB
                pltpu.async_copy(o_hbm.at[pl.ds(boff2, CB)], obufs[s], osems[s])
                pltpu.async_copy(l_hbm.at[pl.ds(boff2, CB)], lbufs[s], lsems[s])
        return acc

    acc = lax.fori_loop(0, NOUTER, outer_body,
                        jnp.zeros((I, F), jnp.float32))
    o_ref[0, :] = jnp.sum(acc, axis=1) * (1.0 / (B * F))


def kernel(out, labels, importance):
    scratch = (
        [pltpu.VMEM((CB, I, F), jnp.float32) for _ in range(2 * NBUF)]
        + [pltpu.SemaphoreType.DMA for _ in range(2 * NBUF)]
    )
    res = pl.pallas_call(
        _ring_kernel,
        in_specs=[
            pl.BlockSpec(memory_space=pl.ANY),
            pl.BlockSpec(memory_space=pl.ANY),
            pl.BlockSpec((I, F), lambda: (0, 0)),
        ],
        out_specs=pl.BlockSpec((1, I), lambda: (0, 0)),
        out_shape=jax.ShapeDtypeStruct((1, I), jnp.float32),
        scratch_shapes=scratch,
    )(out, labels, importance)
    return res[0]


# speedup vs baseline: 1.0058x; 1.0058x over previous
import jax
import jax.numpy as jnp
from jax import lax
from jax.experimental import pallas as pl
from jax.experimental.pallas import tpu as pltpu

B, I, F = 4096, 16, 512
CB = 64
NCHUNK = B // CB
NBUF = 4
NOUTER = NCHUNK // NBUF
assert NCHUNK % NBUF == 0


def _ring_kernel(o_hbm, l_hbm, imp_ref, o_ref, *scr):
    obufs = scr[0:NBUF]
    lbufs = scr[NBUF:2 * NBUF]
    osems = scr[2 * NBUF:3 * NBUF]
    lsems = scr[3 * NBUF:4 * NBUF]

    imp = imp_ref[...]

    for s in range(NBUF):
        boff = s * CB
        pltpu.async_copy(o_hbm.at[pl.ds(boff, CB)], obufs[s], osems[s])
        pltpu.async_copy(l_hbm.at[pl.ds(boff, CB)], lbufs[s], lsems[s])

    def outer_body(c0, acc):
        for s in range(NBUF):
            c = c0 * NBUF + s
            boff = c * CB
            ob, lb = obufs[s], lbufs[s]
            pltpu.make_async_copy(o_hbm.at[pl.ds(boff, CB)], ob, osems[s]).wait()
            pltpu.make_async_copy(l_hbm.at[pl.ds(boff, CB)], lb, lsems[s]).wait()

            def b_body(b, a, ob=ob, lb=lb):
                d = imp * (jnp.abs(lb[b]) - ob[b])
                return a + d * d

            acc = lax.fori_loop(0, CB, b_body, acc, unroll=2)

            @pl.when(c0 < NOUTER - 1)
            def _():
                boff2 = boff + NBUF * CB
                pltpu.async_copy(o_hbm.at[pl.ds(boff2, CB)], obufs[s], osems[s])
                pltpu.async_copy(l_hbm.at[pl.ds(boff2, CB)], lbufs[s], lsems[s])
        return acc

    acc = lax.fori_loop(0, NOUTER, outer_body,
                        jnp.zeros((I, F), jnp.float32))
    o_ref[0, :] = jnp.sum(acc, axis=1) * (1.0 / (B * F))


def kernel(out, labels, importance):
    scratch = (
        [pltpu.VMEM((CB, I, F), jnp.float32) for _ in range(2 * NBUF)]
        + [pltpu.SemaphoreType.DMA for _ in range(2 * NBUF)]
    )
    res = pl.pallas_call(
        _ring_kernel,
        in_specs=[
            pl.BlockSpec(memory_space=pl.ANY),
            pl.BlockSpec(memory_space=pl.ANY),
            pl.BlockSpec((I, F), lambda: (0, 0)),
        ],
        out_specs=pl.BlockSpec((1, I), lambda: (0, 0)),
        out_shape=jax.ShapeDtypeStruct((1, I), jnp.float32),
        scratch_shapes=scratch,
    )(out, labels, importance)
    return res[0]
